# Initial kernel scaffold; baseline (speedup 1.0000x reference)
import jax
import jax.numpy as jnp
from jax.experimental import pallas as pl


def kernel(yi, yi_t):
    raise NotImplementedError("write your pallas kernel here")


# speedup vs baseline: 11.9560x; 11.9560x over previous
import functools

import jax
import jax.numpy as jnp
from jax.experimental import pallas as pl

_T = 0.0025
_M = 1.0
_K = 16
_N = 1024
_BIG = 3.0e38


def _blcd_kernel(yi_ref, yit_ref, e1_ref, e2_ref):
    yi = yi_ref[...]
    yit = yit_ref[...]
    yi_n = yi / jnp.sqrt(jnp.sum(yi * yi, axis=1, keepdims=True) + 1e-12)
    yit_n = yit / jnp.sqrt(jnp.sum(yit * yit, axis=1, keepdims=True) + 1e-12)

    dims = (((1,), (1,)), ((), ()))
    g = jax.lax.dot_general(yi_n, yi_n, dims, preferred_element_type=jnp.float32)
    c = jax.lax.dot_general(yit_n, yi_n, dims, preferred_element_type=jnp.float32)

    d = jnp.sqrt(jnp.maximum(2.0 - 2.0 * g, 0.0) + 1e-12)
    row_iota = jax.lax.broadcasted_iota(jnp.int32, (_N, _N), 0)
    col_iota = jax.lax.broadcasted_iota(jnp.int32, (_N, _N), 1)
    dw = jnp.where(row_iota == col_iota, _BIG, d)

    dsel = []
    csel = []
    for _ in range(_K):
        m = jnp.min(dw, axis=1)
        cand = jnp.where(dw <= m[:, None], col_iota, _N + 1)
        idx = jnp.min(cand, axis=1)
        onehot = col_iota == idx[:, None]
        csel.append(jnp.sum(jnp.where(onehot, c, 0.0), axis=1))
        dsel.append(m)
        dw = jnp.where(onehot, _BIG, dw)

    dsel_a = jnp.stack(dsel, axis=1)
    csel_a = jnp.stack(csel, axis=1)

    dis_yij = 0.5 * dsel_a
    dis_yi_tj = 0.5 * jnp.sqrt(jnp.maximum(2.0 - 2.0 * csel_a, 0.0) + 1e-12)
    out1 = (dis_yij - dis_yi_tj) ** 2 - _T
    e1 = jnp.sum(jnp.where(out1 > 0, out1, 0.0))

    diff = yi_n - yit_n
    dd = 0.5 * jnp.sqrt(jnp.sum(diff * diff, axis=1) + 1e-12)
    out2 = dd + _M - 0.5 * dsel_a[:, 0]
    e2 = jnp.sum(jnp.where(out2 > 0, out2, 0.0))

    e1_ref[...] = jnp.reshape(e1, (1, 1))
    e2_ref[...] = jnp.reshape(e2, (1, 1))


@jax.jit
def kernel(yi, yi_t):
    e1, e2 = pl.pallas_call(
        _blcd_kernel,
        out_shape=[
            jax.ShapeDtypeStruct((1, 1), jnp.float32),
            jax.ShapeDtypeStruct((1, 1), jnp.float32),
        ],
    )(yi, yi_t)
    e1s = e1[0, 0]
    e2s = e2[0, 0]
    return (e1s + e2s, e1s, e2s)
